# baseline (speedup 1.0000x reference)
import jax
import jax.numpy as jnp
from jax import lax
from jax.experimental import pallas as pl
from jax.experimental.pallas import tpu as pltpu
from jax.experimental.pallas import tpu_sc as plsc

D = 64
B = 4096
L = 50
NC = 2
NS = 16
NW = NC * NS
BAGS_PER_W = B // NW
BAGS_PER_CHUNK = 2
IDX_PER_CHUNK = BAGS_PER_CHUNK * L
NCHUNK = BAGS_PER_W // BAGS_PER_CHUNK
LANES = 16
NDG = D // LANES


def _ebag_body(idx_hbm, w_hbm, table_hbm, out_hbm, idx_v, w_v, rows_v, acc_v, sem):
    wid = lax.axis_index("s") * NC + lax.axis_index("c")
    crow = wid * NCHUNK
    pltpu.sync_copy(idx_hbm.at[pl.ds(crow, NCHUNK), :], idx_v)
    pltpu.sync_copy(w_hbm.at[pl.ds(crow, NCHUNK), :], w_v)

    blocks = []
    off = 0
    while off + LANES <= L:
        blocks.append((off, off))
        off += LANES
    if off < L:
        blocks.append((L - LANES, off))

    def chunk_body(c, carry):
        pltpu.async_copy(table_hbm.at[idx_v.at[c]], rows_v, sem).wait()
        for i in range(BAGS_PER_CHUNK):
            base = i * L
            accs = [jnp.zeros((LANES,), jnp.float32) for _ in range(NDG)]
            for load_off, first_l in blocks:
                wv = w_v[c, pl.ds(base + load_off, LANES)]
                for lane in range(first_l - load_off, LANES):
                    l = load_off + lane
                    w = wv[lane]
                    for d in range(NDG):
                        accs[d] = accs[d] + w * rows_v[base + l, pl.ds(d * LANES, LANES)]
            for d in range(NDG):
                acc_v[c * BAGS_PER_CHUNK + i, pl.ds(d * LANES, LANES)] = accs[d]
        return carry

    lax.fori_loop(0, NCHUNK, chunk_body, 0)
    pltpu.sync_copy(acc_v, out_hbm.at[pl.ds(wid * BAGS_PER_W, BAGS_PER_W), :])


def kernel(lookup_tensor, per_sample_weights, table):
    idx = lookup_tensor.reshape(B // BAGS_PER_CHUNK, IDX_PER_CHUNK)
    w = (per_sample_weights * (1.0 / L)).reshape(B // BAGS_PER_CHUNK, IDX_PER_CHUNK)
    mesh = plsc.VectorSubcoreMesh(
        core_axis_name="c", subcore_axis_name="s", num_cores=NC, num_subcores=NS
    )
    f = pl.kernel(
        _ebag_body,
        out_type=jax.ShapeDtypeStruct((B, D), jnp.float32),
        mesh=mesh,
        scratch_types=[
            pltpu.VMEM((NCHUNK, IDX_PER_CHUNK), jnp.int32),
            pltpu.VMEM((NCHUNK, IDX_PER_CHUNK), jnp.float32),
            pltpu.VMEM((IDX_PER_CHUNK, D), jnp.float32),
            pltpu.VMEM((BAGS_PER_W, D), jnp.float32),
            pltpu.SemaphoreType.DMA,
        ],
        compiler_params=pltpu.CompilerParams(use_tc_tiling_on_sc=False),
    )
    return f(idx, w, table)


# speedup vs baseline: 1.0479x; 1.0479x over previous
import jax
import jax.numpy as jnp
from jax import lax
from jax.experimental import pallas as pl
from jax.experimental.pallas import tpu as pltpu
from jax.experimental.pallas import tpu_sc as plsc

D = 64
B = 4096
L = 50
NC = 2
NS = 16
NW = NC * NS
BAGS_PER_W = B // NW
BAGS_PER_CHUNK = 2
IDX_PER_CHUNK = BAGS_PER_CHUNK * L
NCHUNK = BAGS_PER_W // BAGS_PER_CHUNK
LANES = 16
NDG = D // LANES


NBUF = 4


def _ebag_body(idx_hbm, w_hbm, table_hbm, out_hbm, idx_v, w_v, rows_bufs, acc_v, sems):
    wid = lax.axis_index("s") * NC + lax.axis_index("c")
    crow = wid * NCHUNK
    pltpu.sync_copy(idx_hbm.at[pl.ds(crow, NCHUNK), :], idx_v)
    pltpu.sync_copy(w_hbm.at[pl.ds(crow, NCHUNK), :], w_v)

    blocks = []
    off = 0
    while off + LANES <= L:
        blocks.append((off, off))
        off += LANES
    if off < L:
        blocks.append((L - LANES, off))

    def start_gather(c, buf):
        pltpu.async_copy(table_hbm.at[idx_v.at[c]], rows_bufs[buf], sems[buf])

    def wait_gather(c, buf):
        pltpu.make_async_copy(
            table_hbm.at[idx_v.at[c]], rows_bufs[buf], sems[buf]
        ).wait()

    def compute_chunk(c, buf):
        rows_v = rows_bufs[buf]
        for i in range(BAGS_PER_CHUNK):
            base = i * L
            accs = [jnp.zeros((LANES,), jnp.float32) for _ in range(NDG)]
            for load_off, first_l in blocks:
                wv = w_v[c, pl.ds(base + load_off, LANES)]
                for lane in range(first_l - load_off, LANES):
                    l = load_off + lane
                    w = wv[lane]
                    for d in range(NDG):
                        accs[d] = accs[d] + w * rows_v[base + l, pl.ds(d * LANES, LANES)]
            for d in range(NDG):
                acc_v[c * BAGS_PER_CHUNK + i, pl.ds(d * LANES, LANES)] = accs[d]

    for b in range(NBUF - 1):
        start_gather(b, b)

    def outer_body(g, carry):
        for b in range(NBUF):
            c = g * NBUF + b
            nxt = c + NBUF - 1
            nxt_buf = (b + NBUF - 1) % NBUF

            @pl.when(nxt < NCHUNK)
            def _():
                start_gather(nxt, nxt_buf)

            wait_gather(c, b)
            compute_chunk(c, b)
        return carry

    lax.fori_loop(0, NCHUNK // NBUF, outer_body, 0)
    pltpu.sync_copy(acc_v, out_hbm.at[pl.ds(wid * BAGS_PER_W, BAGS_PER_W), :])


def kernel(lookup_tensor, per_sample_weights, table):
    idx = lookup_tensor.reshape(B // BAGS_PER_CHUNK, IDX_PER_CHUNK)
    w = (per_sample_weights * (1.0 / L)).reshape(B // BAGS_PER_CHUNK, IDX_PER_CHUNK)
    mesh = plsc.VectorSubcoreMesh(
        core_axis_name="c", subcore_axis_name="s", num_cores=NC, num_subcores=NS
    )
    f = pl.kernel(
        _ebag_body,
        out_type=jax.ShapeDtypeStruct((B, D), jnp.float32),
        mesh=mesh,
        scratch_types=[
            pltpu.VMEM((NCHUNK, IDX_PER_CHUNK), jnp.int32),
            pltpu.VMEM((NCHUNK, IDX_PER_CHUNK), jnp.float32),
            [pltpu.VMEM((IDX_PER_CHUNK, D), jnp.float32) for _ in range(NBUF)],
            pltpu.VMEM((BAGS_PER_W, D), jnp.float32),
            [pltpu.SemaphoreType.DMA for _ in range(NBUF)],
        ],
        compiler_params=pltpu.CompilerParams(use_tc_tiling_on_sc=False),
    )
    return f(idx, w, table)
